# Initial kernel scaffold; baseline (speedup 1.0000x reference)
import jax
import jax.numpy as jnp
from jax.experimental import pallas as pl


def kernel(x, edge_index, lin0_W, lin0_b, conv_W, lin1_W, lin1_b):
    raise NotImplementedError("write your pallas kernel here")


# speedup vs baseline: 3.3044x; 3.3044x over previous
import functools
import math

import jax
import jax.numpy as jnp
from jax import lax
from jax.experimental import pallas as pl
from jax.experimental.pallas import tpu as pltpu
from jax.experimental.pallas import tpu_sc as plsc

N = 10000
E = 320000
D = 128
C = 40
ALPHA = 0.1
THETA = 0.5

NC = 2
NS = 16
NW = NC * NS

CH = 128
PT = 80
EP = NW * PT * CH
NP = 10112
ZR = NP // NS

_mesh = plsc.VectorSubcoreMesh(core_axis_name="c", subcore_axis_name="s")


@functools.partial(
    pl.kernel,
    out_type=jax.ShapeDtypeStruct((NC, NP, D), jnp.float32),
    mesh=_mesh,
    scratch_types=[
        pltpu.VMEM((PT, CH), jnp.int32),
        pltpu.VMEM((PT, CH), jnp.int32),
        pltpu.VMEM((CH, D), jnp.float32),
        pltpu.VMEM_SHARED((NP, D), jnp.float32),
        pltpu.SemaphoreType.DMA,
    ],
)
def _sc_prop(h_hbm, src_hbm, dst_hbm, zeros_hbm, out_hbm,
             src_v, dst_v, rows_v, agg_s, sem):
    cid = lax.axis_index("c")
    sid = lax.axis_index("s")
    wid = cid * NS + sid
    pltpu.sync_copy(zeros_hbm, agg_s.at[pl.ds(sid * ZR, ZR)])
    pltpu.sync_copy(src_hbm.at[wid], src_v)
    pltpu.sync_copy(dst_hbm.at[wid], dst_v)
    plsc.subcore_barrier()

    def body(g, carry):
        pltpu.async_copy(h_hbm.at[src_v.at[g]], rows_v, sem).wait()
        pltpu.sync_copy(rows_v, agg_s.at[dst_v.at[g]], add=True)
        return carry

    lax.fori_loop(0, PT, body, 0)
    plsc.subcore_barrier()
    pltpu.sync_copy(agg_s.at[pl.ds(sid * ZR, ZR)],
                    out_hbm.at[cid, pl.ds(sid * ZR, ZR)])


def _lin0_body(x_ref, w_ref, b_ref, h_ref):
    h = jnp.dot(x_ref[...], w_ref[...], preferred_element_type=jnp.float32)
    h_ref[pl.ds(0, N), :] = jnp.maximum(h + b_ref[...], 0.0)
    h_ref[pl.ds(N, NP - N), :] = jnp.zeros((NP - N, D), jnp.float32)


def _mix_body(p_ref, x0_ref, w_ref, out_ref, *, beta):
    agg = p_ref[0, pl.ds(0, N), :] + p_ref[1, pl.ds(0, N), :]
    hmix = (1.0 - ALPHA) * agg + ALPHA * x0_ref[pl.ds(0, N), :]
    xw = jnp.dot(hmix, w_ref[...], preferred_element_type=jnp.float32)
    out_ref[pl.ds(0, N), :] = jnp.maximum((1.0 - beta) * hmix + beta * xw, 0.0)
    out_ref[pl.ds(N, NP - N), :] = jnp.zeros((NP - N, D), jnp.float32)


def _final_body(p_ref, x0_ref, w_ref, w1_ref, b1_ref, out_ref, *, beta):
    agg = p_ref[0, pl.ds(0, N), :] + p_ref[1, pl.ds(0, N), :]
    hmix = (1.0 - ALPHA) * agg + ALPHA * x0_ref[pl.ds(0, N), :]
    xw = jnp.dot(hmix, w_ref[...], preferred_element_type=jnp.float32)
    xcur = jnp.maximum((1.0 - beta) * hmix + beta * xw, 0.0)
    o = jnp.dot(xcur, w1_ref[...], preferred_element_type=jnp.float32)
    o = o + b1_ref[...]
    m = jnp.max(o, axis=-1, keepdims=True)
    lse = m + jnp.log(jnp.sum(jnp.exp(o - m), axis=-1, keepdims=True))
    out_ref[...] = o - lse


def kernel(x, edge_index, lin0_W, lin0_b, conv_W, lin1_W, lin1_b):
    src = jnp.concatenate(
        [edge_index[0], jnp.full((EP - E,), N, jnp.int32)]).reshape(NW, PT, CH)
    dst = jnp.concatenate(
        [edge_index[1], jnp.full((EP - E,), N, jnp.int32)]).reshape(NW, PT, CH)
    zeros = jnp.zeros((ZR, D), jnp.float32)
    w1p = jnp.zeros((D, D), jnp.float32).at[:, :C].set(lin1_W)
    b1p = jnp.full((1, D), -1e30, jnp.float32).at[0, :C].set(lin1_b)

    h = pl.pallas_call(
        _lin0_body,
        out_shape=jax.ShapeDtypeStruct((NP, D), jnp.float32),
    )(x, lin0_W, lin0_b.reshape(1, D))

    p0 = _sc_prop(h, src, dst, zeros)

    beta0 = math.log(THETA / 1.0 + 1.0)
    x1 = pl.pallas_call(
        functools.partial(_mix_body, beta=beta0),
        out_shape=jax.ShapeDtypeStruct((NP, D), jnp.float32),
    )(p0, h, conv_W[0])

    p1 = _sc_prop(x1, src, dst, zeros)

    beta1 = math.log(THETA / 2.0 + 1.0)
    out = pl.pallas_call(
        functools.partial(_final_body, beta=beta1),
        out_shape=jax.ShapeDtypeStruct((N, D), jnp.float32),
    )(p1, h, conv_W[1], w1p, b1p)

    return out[:, :C]


# speedup vs baseline: 3.7741x; 1.1421x over previous
import functools
import math

import jax
import jax.numpy as jnp
from jax import lax
from jax.experimental import pallas as pl
from jax.experimental.pallas import tpu as pltpu
from jax.experimental.pallas import tpu_sc as plsc

N = 10000
E = 320000
D = 128
C = 40
ALPHA = 0.1
THETA = 0.5

NC = 2
NS = 16
NW = NC * NS

CH = 128
PT = 80
HT = PT // 2
EP = NW * PT * CH
NP = 10112
ZR = NP // NS

_mesh = plsc.VectorSubcoreMesh(core_axis_name="c", subcore_axis_name="s")


@functools.partial(
    pl.kernel,
    out_type=jax.ShapeDtypeStruct((NC, NP, D), jnp.float32),
    mesh=_mesh,
    scratch_types=[
        pltpu.VMEM((HT, CH), jnp.int32),
        pltpu.VMEM((HT, CH), jnp.int32),
        pltpu.VMEM((CH, D), jnp.float32),
        pltpu.VMEM((CH, D), jnp.float32),
        pltpu.VMEM_SHARED((NP, D), jnp.float32),
        pltpu.SemaphoreType.DMA,
        pltpu.SemaphoreType.DMA,
    ],
)
def _sc_prop(h_hbm, src_hbm, dst_hbm, zeros_hbm, out_hbm,
             src_v, dst_v, rows_a, rows_b, agg_s, sem_a, sem_b):
    cid = lax.axis_index("c")
    sid = lax.axis_index("s")
    wid = cid * NS + sid
    pltpu.sync_copy(zeros_hbm, agg_s.at[pl.ds(sid * ZR, ZR)])
    plsc.subcore_barrier()

    def fire(g, buf, sem):
        pltpu.async_copy(h_hbm.at[src_v.at[jnp.minimum(g, HT - 1)]],
                         buf, sem)

    def drain(buf, sem):
        pltpu.make_async_copy(h_hbm.at[src_v.at[0]], buf, sem).wait()

    def scat(g, buf):
        pltpu.sync_copy(buf, agg_s.at[dst_v.at[g]], add=True)

    for hh in range(2):
        pltpu.sync_copy(src_hbm.at[wid, pl.ds(hh * HT, HT)], src_v)
        pltpu.sync_copy(dst_hbm.at[wid, pl.ds(hh * HT, HT)], dst_v)
        fire(0, rows_a, sem_a)

        def body(i, carry):
            g = i * 2
            fire(g + 1, rows_b, sem_b)
            drain(rows_a, sem_a)
            scat(g, rows_a)
            fire(g + 2, rows_a, sem_a)
            drain(rows_b, sem_b)
            scat(g + 1, rows_b)
            return carry

        lax.fori_loop(0, HT // 2, body, 0)
        drain(rows_a, sem_a)
    plsc.subcore_barrier()
    pltpu.sync_copy(agg_s.at[pl.ds(sid * ZR, ZR)],
                    out_hbm.at[cid, pl.ds(sid * ZR, ZR)])


def _lin0_body(x_ref, w_ref, b_ref, h_ref):
    h = jnp.dot(x_ref[...], w_ref[...], preferred_element_type=jnp.float32)
    h_ref[pl.ds(0, N), :] = jnp.maximum(h + b_ref[...], 0.0)
    h_ref[pl.ds(N, NP - N), :] = jnp.zeros((NP - N, D), jnp.float32)


def _mix_body(p_ref, x0_ref, w_ref, out_ref, *, beta):
    agg = p_ref[0, pl.ds(0, N), :] + p_ref[1, pl.ds(0, N), :]
    hmix = (1.0 - ALPHA) * agg + ALPHA * x0_ref[pl.ds(0, N), :]
    xw = jnp.dot(hmix, w_ref[...], preferred_element_type=jnp.float32)
    out_ref[pl.ds(0, N), :] = jnp.maximum((1.0 - beta) * hmix + beta * xw, 0.0)
    out_ref[pl.ds(N, NP - N), :] = jnp.zeros((NP - N, D), jnp.float32)


def _final_body(p_ref, x0_ref, w_ref, w1_ref, b1_ref, out_ref, *, beta):
    agg = p_ref[0, pl.ds(0, N), :] + p_ref[1, pl.ds(0, N), :]
    hmix = (1.0 - ALPHA) * agg + ALPHA * x0_ref[pl.ds(0, N), :]
    xw = jnp.dot(hmix, w_ref[...], preferred_element_type=jnp.float32)
    xcur = jnp.maximum((1.0 - beta) * hmix + beta * xw, 0.0)
    o = jnp.dot(xcur, w1_ref[...], preferred_element_type=jnp.float32)
    o = o + b1_ref[...]
    m = jnp.max(o, axis=-1, keepdims=True)
    lse = m + jnp.log(jnp.sum(jnp.exp(o - m), axis=-1, keepdims=True))
    out_ref[...] = o - lse


def kernel(x, edge_index, lin0_W, lin0_b, conv_W, lin1_W, lin1_b):
    src = jnp.concatenate(
        [edge_index[0], jnp.full((EP - E,), N, jnp.int32)]).reshape(NW, PT, CH)
    dst = jnp.concatenate(
        [edge_index[1], jnp.full((EP - E,), N, jnp.int32)]).reshape(NW, PT, CH)
    zeros = jnp.zeros((ZR, D), jnp.float32)
    w1p = jnp.zeros((D, D), jnp.float32).at[:, :C].set(lin1_W)
    b1p = jnp.full((1, D), -1e30, jnp.float32).at[0, :C].set(lin1_b)

    h = pl.pallas_call(
        _lin0_body,
        out_shape=jax.ShapeDtypeStruct((NP, D), jnp.float32),
    )(x, lin0_W, lin0_b.reshape(1, D))

    p0 = _sc_prop(h, src, dst, zeros)

    beta0 = math.log(THETA / 1.0 + 1.0)
    x1 = pl.pallas_call(
        functools.partial(_mix_body, beta=beta0),
        out_shape=jax.ShapeDtypeStruct((NP, D), jnp.float32),
    )(p0, h, conv_W[0])

    p1 = _sc_prop(x1, src, dst, zeros)

    beta1 = math.log(THETA / 2.0 + 1.0)
    out = pl.pallas_call(
        functools.partial(_final_body, beta=beta1),
        out_shape=jax.ShapeDtypeStruct((N, D), jnp.float32),
    )(p1, h, conv_W[1], w1p, b1p)

    return out[:, :C]


# speedup vs baseline: 4.4310x; 1.1741x over previous
import functools
import math

import numpy as np

import jax
import jax.numpy as jnp
from jax import lax
from jax.experimental import pallas as pl
from jax.experimental.pallas import tpu as pltpu
from jax.experimental.pallas import tpu_sc as plsc

N = 10000
E = 320000
D = 128
C = 40
ALPHA = 0.1
THETA = 0.5

NC = 2
NS = 16
NW = NC * NS

CH = 128
PT = 80
HT = PT // 4
EP = NW * PT * CH
NP = 10112
ZR = NP // NS

_mesh = plsc.VectorSubcoreMesh(core_axis_name="c", subcore_axis_name="s")

_PERM = np.arange(D).reshape(4, 2, 16).transpose(0, 2, 1).reshape(D)


@functools.partial(
    pl.kernel,
    out_type=jax.ShapeDtypeStruct((NC, NP, D), jnp.float32),
    mesh=_mesh,
    compiler_params=pltpu.CompilerParams(use_tc_tiling_on_sc=False),
    scratch_types=[
        pltpu.VMEM((HT, CH), jnp.int32),
        pltpu.VMEM((HT, CH), jnp.int32),
        pltpu.VMEM((CH, D // 2), jnp.int32),
        pltpu.VMEM((CH, D // 2), jnp.int32),
        pltpu.VMEM((CH, D), jnp.float32),
        pltpu.VMEM_SHARED((NP, D), jnp.float32),
        pltpu.SemaphoreType.DMA,
        pltpu.SemaphoreType.DMA,
    ],
)
def _sc_prop(hb_hbm, src_hbm, dst_hbm, zeros_hbm, out_hbm,
             src_v, dst_v, rows_a, rows_b, rows_f, agg_s, sem_a, sem_b):
    cid = lax.axis_index("c")
    sid = lax.axis_index("s")
    wid = cid * NS + sid
    pltpu.sync_copy(zeros_hbm, agg_s.at[pl.ds(sid * ZR, ZR)])
    plsc.subcore_barrier()

    def fire(g, buf, sem):
        pltpu.async_copy(hb_hbm.at[src_v.at[jnp.minimum(g, HT - 1)]],
                         buf, sem)

    def drain(buf, sem):
        pltpu.make_async_copy(hb_hbm.at[src_v.at[0]], buf, sem).wait()

    def conv(buf):
        def crow(r8, carry):
            for rr in range(8):
                r = r8 * 8 + rr
                for g4 in range(4):
                    v = buf[r, pl.ds(g4 * 16, 16)]
                    a = lax.bitcast_convert_type(v << 16, jnp.float32)
                    b = lax.bitcast_convert_type(v & jnp.int32(-65536),
                                                 jnp.float32)
                    rows_f[r, pl.ds(g4 * 32, 16)] = a
                    rows_f[r, pl.ds(g4 * 32 + 16, 16)] = b
            return carry

        lax.fori_loop(0, CH // 8, crow, 0)

    def scat(g):
        pltpu.sync_copy(rows_f, agg_s.at[dst_v.at[g]], add=True)

    for hh in range(4):
        pltpu.sync_copy(src_hbm.at[wid, pl.ds(hh * HT, HT)], src_v)
        pltpu.sync_copy(dst_hbm.at[wid, pl.ds(hh * HT, HT)], dst_v)
        fire(0, rows_a, sem_a)

        def body(i, carry):
            g = i * 2
            fire(g + 1, rows_b, sem_b)
            drain(rows_a, sem_a)
            conv(rows_a)
            fire(g + 2, rows_a, sem_a)
            scat(g)
            drain(rows_b, sem_b)
            conv(rows_b)
            scat(g + 1)
            return carry

        lax.fori_loop(0, HT // 2, body, 0)
        drain(rows_a, sem_a)
    plsc.subcore_barrier()
    pltpu.sync_copy(agg_s.at[pl.ds(sid * ZR, ZR)],
                    out_hbm.at[cid, pl.ds(sid * ZR, ZR)])


def _lin0_body(x_ref, w_ref, b_ref, h_ref):
    h = jnp.dot(x_ref[...], w_ref[...], preferred_element_type=jnp.float32)
    h_ref[pl.ds(0, N), :] = jnp.maximum(h + b_ref[...], 0.0)
    h_ref[pl.ds(N, NP - N), :] = jnp.zeros((NP - N, D), jnp.float32)


def _mix_body(p_ref, x0_ref, w_ref, out_ref, *, beta):
    agg = p_ref[0, pl.ds(0, N), :] + p_ref[1, pl.ds(0, N), :]
    hmix = (1.0 - ALPHA) * agg + ALPHA * x0_ref[pl.ds(0, N), :]
    xw = jnp.dot(hmix, w_ref[...], preferred_element_type=jnp.float32)
    out_ref[pl.ds(0, N), :] = jnp.maximum((1.0 - beta) * hmix + beta * xw, 0.0)
    out_ref[pl.ds(N, NP - N), :] = jnp.zeros((NP - N, D), jnp.float32)


def _final_body(p_ref, x0_ref, w_ref, w1_ref, b1_ref, out_ref, *, beta):
    agg = p_ref[0, pl.ds(0, N), :] + p_ref[1, pl.ds(0, N), :]
    hmix = (1.0 - ALPHA) * agg + ALPHA * x0_ref[pl.ds(0, N), :]
    xw = jnp.dot(hmix, w_ref[...], preferred_element_type=jnp.float32)
    xcur = jnp.maximum((1.0 - beta) * hmix + beta * xw, 0.0)
    o = jnp.dot(xcur, w1_ref[...], preferred_element_type=jnp.float32)
    o = o + b1_ref[...]
    m = jnp.max(o, axis=-1, keepdims=True)
    lse = m + jnp.log(jnp.sum(jnp.exp(o - m), axis=-1, keepdims=True))
    out_ref[...] = o - lse


def _pack_table(hf):
    hb = hf[:, _PERM].astype(jnp.bfloat16)
    return lax.bitcast_convert_type(hb.reshape(NP, D // 2, 2), jnp.int32)


def kernel(x, edge_index, lin0_W, lin0_b, conv_W, lin1_W, lin1_b):
    src = jnp.concatenate(
        [edge_index[0], jnp.full((EP - E,), N, jnp.int32)]).reshape(NW, PT, CH)
    dst = jnp.concatenate(
        [edge_index[1], jnp.full((EP - E,), N, jnp.int32)]).reshape(NW, PT, CH)
    zeros = jnp.zeros((ZR, D), jnp.float32)
    w1p = jnp.zeros((D, D), jnp.float32).at[:, :C].set(lin1_W)
    b1p = jnp.full((1, D), -1e30, jnp.float32).at[0, :C].set(lin1_b)

    h = pl.pallas_call(
        _lin0_body,
        out_shape=jax.ShapeDtypeStruct((NP, D), jnp.float32),
    )(x, lin0_W, lin0_b.reshape(1, D))

    p0 = _sc_prop(_pack_table(h), src, dst, zeros)

    beta0 = math.log(THETA / 1.0 + 1.0)
    x1 = pl.pallas_call(
        functools.partial(_mix_body, beta=beta0),
        out_shape=jax.ShapeDtypeStruct((NP, D), jnp.float32),
    )(p0, h, conv_W[0])

    p1 = _sc_prop(_pack_table(x1), src, dst, zeros)

    beta1 = math.log(THETA / 2.0 + 1.0)
    out = pl.pallas_call(
        functools.partial(_final_body, beta=beta1),
        out_shape=jax.ShapeDtypeStruct((N, D), jnp.float32),
    )(p1, h, conv_W[1], w1p, b1p)

    return out[:, :C]


# speedup vs baseline: 7.4774x; 1.6875x over previous
import functools
import math

import jax
import jax.numpy as jnp
from jax import lax
from jax.experimental import pallas as pl
from jax.experimental.pallas import tpu as pltpu
from jax.experimental.pallas import tpu_sc as plsc

N = 10000
E = 320000
D = 128
DH = D // 2
C = 40
ALPHA = 0.1
THETA = 0.5

NC = 2
NS = 16

CH = 128
PT = 160
HT = PT // 4
EP = NS * PT * CH
NP = 10112
ZR = NP // NS

_mesh = plsc.VectorSubcoreMesh(core_axis_name="c", subcore_axis_name="s")


@functools.partial(
    pl.kernel,
    out_type=jax.ShapeDtypeStruct((NC, NP, DH), jnp.float32),
    mesh=_mesh,
    compiler_params=pltpu.CompilerParams(use_tc_tiling_on_sc=False),
    scratch_types=[
        pltpu.VMEM((HT, CH), jnp.int32),
        pltpu.VMEM((HT, CH), jnp.int32),
        pltpu.VMEM((CH, DH), jnp.float32),
        pltpu.VMEM((CH, DH), jnp.float32),
        pltpu.VMEM_SHARED((NP, DH), jnp.float32),
        pltpu.VMEM_SHARED((NP, DH), jnp.float32),
        pltpu.SemaphoreType.DMA,
        pltpu.SemaphoreType.DMA,
    ],
)
def _sc_prop(h2_hbm, src_hbm, dst_hbm, zeros_hbm, out_hbm,
             src_v, dst_v, rows_a, rows_b, h_s, agg_s, sem_a, sem_b):
    cid = lax.axis_index("c")
    sid = lax.axis_index("s")
    stripe = pl.ds(sid * ZR, ZR)
    pltpu.sync_copy(zeros_hbm, agg_s.at[stripe])
    pltpu.sync_copy(h2_hbm.at[cid, stripe], h_s.at[stripe])
    plsc.subcore_barrier()

    def fire(g, buf, sem):
        pltpu.async_copy(h_s.at[src_v.at[jnp.minimum(g, HT - 1)]],
                         buf, sem)

    def drain(buf, sem):
        pltpu.make_async_copy(h_s.at[src_v.at[0]], buf, sem).wait()

    def scat(g, buf):
        pltpu.sync_copy(buf, agg_s.at[dst_v.at[g]], add=True)

    for hh in range(4):
        pltpu.sync_copy(src_hbm.at[sid, pl.ds(hh * HT, HT)], src_v)
        pltpu.sync_copy(dst_hbm.at[sid, pl.ds(hh * HT, HT)], dst_v)
        fire(0, rows_a, sem_a)

        def body(i, carry):
            g = i * 2
            fire(g + 1, rows_b, sem_b)
            drain(rows_a, sem_a)
            scat(g, rows_a)
            fire(g + 2, rows_a, sem_a)
            drain(rows_b, sem_b)
            scat(g + 1, rows_b)
            return carry

        lax.fori_loop(0, HT // 2, body, 0)
        drain(rows_a, sem_a)
    plsc.subcore_barrier()
    pltpu.sync_copy(agg_s.at[stripe], out_hbm.at[cid, stripe])


def _split_store(out_ref, res):
    out_ref[0, pl.ds(0, N), :] = res[:, :DH]
    out_ref[1, pl.ds(0, N), :] = res[:, DH:]
    z = jnp.zeros((NP - N, DH), jnp.float32)
    out_ref[0, pl.ds(N, NP - N), :] = z
    out_ref[1, pl.ds(N, NP - N), :] = z


def _lin0_body(x_ref, w_ref, b_ref, h_ref):
    h = jnp.dot(x_ref[...], w_ref[...], preferred_element_type=jnp.float32)
    _split_store(h_ref, jnp.maximum(h + b_ref[...], 0.0))


def _cat(ref2):
    return jnp.concatenate(
        [ref2[0, pl.ds(0, N), :], ref2[1, pl.ds(0, N), :]], axis=1)


def _mix_body(p_ref, x0_ref, w_ref, out_ref, *, beta):
    hmix = (1.0 - ALPHA) * _cat(p_ref) + ALPHA * _cat(x0_ref)
    xw = jnp.dot(hmix, w_ref[...], preferred_element_type=jnp.float32)
    _split_store(out_ref, jnp.maximum((1.0 - beta) * hmix + beta * xw, 0.0))


def _final_body(p_ref, x0_ref, w_ref, w1_ref, b1_ref, out_ref, *, beta):
    hmix = (1.0 - ALPHA) * _cat(p_ref) + ALPHA * _cat(x0_ref)
    xw = jnp.dot(hmix, w_ref[...], preferred_element_type=jnp.float32)
    xcur = jnp.maximum((1.0 - beta) * hmix + beta * xw, 0.0)
    o = jnp.dot(xcur, w1_ref[...], preferred_element_type=jnp.float32)
    o = o + b1_ref[...]
    m = jnp.max(o, axis=-1, keepdims=True)
    lse = m + jnp.log(jnp.sum(jnp.exp(o - m), axis=-1, keepdims=True))
    out_ref[...] = o - lse


def kernel(x, edge_index, lin0_W, lin0_b, conv_W, lin1_W, lin1_b):
    src = jnp.concatenate(
        [edge_index[0], jnp.full((EP - E,), N, jnp.int32)]).reshape(NS, PT, CH)
    dst = jnp.concatenate(
        [edge_index[1], jnp.full((EP - E,), N, jnp.int32)]).reshape(NS, PT, CH)
    zeros = jnp.zeros((ZR, DH), jnp.float32)
    w1p = jnp.zeros((D, D), jnp.float32).at[:, :C].set(lin1_W)
    b1p = jnp.full((1, D), -1e30, jnp.float32).at[0, :C].set(lin1_b)

    h = pl.pallas_call(
        _lin0_body,
        out_shape=jax.ShapeDtypeStruct((NC, NP, DH), jnp.float32),
    )(x, lin0_W, lin0_b.reshape(1, D))

    p0 = _sc_prop(h, src, dst, zeros)

    beta0 = math.log(THETA / 1.0 + 1.0)
    x1 = pl.pallas_call(
        functools.partial(_mix_body, beta=beta0),
        out_shape=jax.ShapeDtypeStruct((NC, NP, DH), jnp.float32),
    )(p0, h, conv_W[0])

    p1 = _sc_prop(x1, src, dst, zeros)

    beta1 = math.log(THETA / 2.0 + 1.0)
    out = pl.pallas_call(
        functools.partial(_final_body, beta=beta1),
        out_shape=jax.ShapeDtypeStruct((N, D), jnp.float32),
    )(p1, h, conv_W[1], w1p, b1p)

    return out[:, :C]
